# baseline (speedup 1.0000x reference)
import functools

import jax
import jax.numpy as jnp
from jax import lax
from jax.experimental import pallas as pl
from jax.experimental.pallas import tpu as pltpu
from jax.experimental.pallas import tpu_sc as plsc

NC = 2
NS = 16
NW = NC * NS
LANE = 16

_MESH = plsc.VectorSubcoreMesh(core_axis_name="c", subcore_axis_name="s")
_F32 = jnp.float32
_SC_PARAMS = pltpu.CompilerParams(
    needs_layout_passes=False, use_tc_tiling_on_sc=False)


def _degree_partials(dst_flat, n_nodes):
    e = dst_flat.shape[0]
    per_w = e // NW

    @functools.partial(
        pl.kernel,
        mesh=_MESH,
        out_type=jax.ShapeDtypeStruct((NW, n_nodes), _F32),
        compiler_params=_SC_PARAMS,
        scratch_types=[
            pltpu.VMEM((per_w,), jnp.int32),
            pltpu.VMEM((n_nodes,), _F32),
        ],
    )
    def deg_kernel(dst_hbm, out_hbm, idx_v, acc_v):
        cid = lax.axis_index("c")
        sid = lax.axis_index("s")
        wid = sid * NC + cid
        pltpu.sync_copy(dst_hbm.at[pl.ds(wid * per_w, per_w)], idx_v)
        zero = jnp.zeros((LANE,), _F32)

        def zbody(i, c):
            acc_v[pl.ds(i * LANE, LANE)] = zero
            return c

        lax.fori_loop(0, n_nodes // LANE, zbody, 0)
        ones = jnp.ones((LANE,), _F32)

        def body(i, c):
            idx = idx_v[pl.ds(i * LANE, LANE)]
            plsc.addupdate_scatter(acc_v, [idx], ones)
            return c

        lax.fori_loop(0, per_w // LANE, body, 0)
        pltpu.sync_copy(acc_v, out_hbm.at[wid])

    return deg_kernel(dst_flat)


_AGG_C = 80
_AGG_NB = 5


def _edge_aggregate(g2, src4, dst3):
    n2, hh = g2.shape
    n = n2 // NC
    _, _, chw, c = src4.shape
    rows_t = n // NS
    zrows = rows_t // 5

    @functools.partial(
        pl.kernel,
        mesh=_MESH,
        out_type=jax.ShapeDtypeStruct((NC, n, hh), _F32),
        compiler_params=_SC_PARAMS,
        scratch_types=[
            pltpu.VMEM((chw, c), jnp.int32),
            pltpu.VMEM((chw, c), jnp.int32),
            pltpu.VMEM((_AGG_NB, c, hh), _F32),
            pltpu.VMEM((zrows, hh), _F32),
            pltpu.VMEM_SHARED((n, hh), _F32),
            pltpu.SemaphoreType.DMA,
            pltpu.SemaphoreType.DMA,
            pltpu.SemaphoreType.DMA,
            pltpu.SemaphoreType.DMA,
            pltpu.SemaphoreType.DMA,
        ],
    )
    def agg_kernel(g_hbm, src_hbm, dst_hbm, out_hbm,
                   idx_s, idx_d, buf, zbuf, acc_sh, s0, s1, s2, s3, s4):
        sems = (s0, s1, s2, s3, s4)
        cid = lax.axis_index("c")
        sid = lax.axis_index("s")
        pltpu.sync_copy(src_hbm.at[cid, sid], idx_s)
        pltpu.sync_copy(dst_hbm.at[sid], idx_d)

        zero = jnp.zeros((LANE,), _F32)
        ncol = hh // LANE

        def zbody(i, cc):
            zbuf[i // ncol, pl.ds((i % ncol) * LANE, LANE)] = zero
            return cc

        lax.fori_loop(0, zrows * ncol, zbody, 0)
        for k in range(5):
            pltpu.sync_copy(zbuf, acc_sh.at[pl.ds(sid * rows_t + k * zrows, zrows)])
        plsc.subcore_barrier()

        for b in range(_AGG_NB):
            pltpu.async_copy(g_hbm.at[idx_s.at[b]], buf.at[b], sems[b])

        def outer(j0, cc):
            for b in range(_AGG_NB):
                j = j0 * _AGG_NB + b
                pltpu.make_async_copy(g_hbm.at[idx_s.at[j]], buf.at[b], sems[b]).wait()
                pltpu.sync_copy(buf.at[b], acc_sh.at[idx_d.at[j]], add=True)

                @pl.when(j + _AGG_NB < chw)
                def _():
                    pltpu.async_copy(g_hbm.at[idx_s.at[j + _AGG_NB]], buf.at[b], sems[b])
            return cc

        lax.fori_loop(0, chw // _AGG_NB, outer, 0)
        plsc.subcore_barrier()
        for k in range(5):
            off = sid * rows_t + k * zrows
            pltpu.sync_copy(acc_sh.at[pl.ds(off, zrows)],
                            out_hbm.at[cid, pl.ds(off, zrows)])

    return agg_kernel(g2, src4, dst3)


_CMB_C = 40
_CMB_NB = 5


def _edge_combine(a2, b2, src4, dst4):
    n2, hh = a2.shape
    _, _, chw, c = src4.shape
    per_t = chw * c
    e2 = NS * per_t // 2
    c2 = c // 2
    h = NC * hh

    @functools.partial(
        pl.kernel,
        mesh=_MESH,
        out_type=jax.ShapeDtypeStruct((NC, e2, h), _F32),
        compiler_params=_SC_PARAMS,
        scratch_types=[
            pltpu.VMEM((chw, c), jnp.int32),
            pltpu.VMEM((chw, c), jnp.int32),
            pltpu.VMEM((_CMB_NB, c, hh), _F32),
            pltpu.VMEM((_CMB_NB, c, hh), _F32),
            pltpu.VMEM((_CMB_NB, c2, h), _F32),
            pltpu.SemaphoreType.DMA,
            pltpu.SemaphoreType.DMA,
            pltpu.SemaphoreType.DMA,
            pltpu.SemaphoreType.DMA,
            pltpu.SemaphoreType.DMA,
            pltpu.SemaphoreType.DMA,
            pltpu.SemaphoreType.DMA,
            pltpu.SemaphoreType.DMA,
            pltpu.SemaphoreType.DMA,
            pltpu.SemaphoreType.DMA,
        ],
    )
    def cmb_kernel(a_hbm, b_hbm, src_hbm, dst_hbm, out_hbm,
                   idx_s, idx_d, buf_a, buf_b, buf_o,
                   g0, g1, g2, g3, g4, o0, o1, o2, o3, o4):
        gsem = (g0, g1, g2, g3, g4)
        osem = (o0, o1, o2, o3, o4)
        cid = lax.axis_index("c")
        sid = lax.axis_index("s")
        pltpu.sync_copy(src_hbm.at[cid, sid], idx_s)
        pltpu.sync_copy(dst_hbm.at[cid, sid], idx_d)

        for b in range(_CMB_NB):
            pltpu.async_copy(a_hbm.at[idx_s.at[b]], buf_a.at[b], gsem[b])
            pltpu.async_copy(b_hbm.at[idx_d.at[b]], buf_b.at[b], gsem[b])

        ncol = hh // LANE

        def outer(j0, cc):
            for b in range(_CMB_NB):
                j = j0 * _CMB_NB + b
                out_row = (sid * per_t + j * c) // 2
                pltpu.make_async_copy(a_hbm.at[idx_s.at[j]], buf_a.at[b], gsem[b]).wait()
                pltpu.make_async_copy(b_hbm.at[idx_d.at[j]], buf_b.at[b], gsem[b]).wait()

                @pl.when(j >= _CMB_NB)
                def _():
                    pltpu.make_async_copy(
                        buf_o.at[b],
                        out_hbm.at[cid, pl.ds(out_row - _CMB_NB * c2, c2)],
                        osem[b]).wait()

                def addrow(rr, carry, _b=b):
                    for col in range(2 * ncol):
                        sr = rr + (c2 if col >= ncol else 0)
                        sc = (col % ncol) * LANE
                        buf_o[_b, rr, pl.ds(col * LANE, LANE)] = (
                            buf_a[_b, sr, pl.ds(sc, LANE)]
                            + buf_b[_b, sr, pl.ds(sc, LANE)])
                    return carry

                lax.fori_loop(0, c2, addrow, 0)
                pltpu.async_copy(buf_o.at[b], out_hbm.at[cid, pl.ds(out_row, c2)],
                                 osem[b])

                @pl.when(j + _CMB_NB < chw)
                def _():
                    pltpu.async_copy(a_hbm.at[idx_s.at[j + _CMB_NB]], buf_a.at[b], gsem[b])
                    pltpu.async_copy(b_hbm.at[idx_d.at[j + _CMB_NB]], buf_b.at[b], gsem[b])
            return cc

        lax.fori_loop(0, chw // _CMB_NB, outer, 0)
        for b in range(_CMB_NB):
            j = chw - _CMB_NB + b
            out_row = (sid * per_t + j * c) // 2
            pltpu.make_async_copy(buf_o.at[b], out_hbm.at[cid, pl.ds(out_row, c2)],
                                  osem[b]).wait()

    return cmb_kernel(a2, b2, src4, dst4)


_BK = 2000
_BE = 4000


def _dot(x, w):
    return jnp.dot(x, w, preferred_element_type=_F32)


def _split(t, hh):
    return jnp.stack([t[:, :hh], t[:, hh:]], axis=0)


def _tc_transform(x, hist, wt, bt, w0):
    n, d = x.shape
    h = wt.shape[1]
    hh = h // NC
    grid = (n // _BK,)

    def body(x_ref, hist_ref, wt_ref, bt_ref, w0_ref, g0_ref, dinv_ref):
        deg = 1.0 + jnp.sum(hist_ref[...], axis=1)
        dinv = lax.rsqrt(deg)[:, None]
        h0 = jnp.maximum(_dot(x_ref[...], wt_ref[...]) + bt_ref[...], 0.0)
        g0_ref[...] = _split(_dot(h0, w0_ref[...]) * dinv, hh)
        dinv_ref[...] = dinv

    return pl.pallas_call(
        body,
        grid=grid,
        in_specs=[
            pl.BlockSpec((_BK, d), lambda i: (i, 0)),
            pl.BlockSpec((_BK, NW), lambda i: (i, 0)),
            pl.BlockSpec((d, h), lambda i: (0, 0)),
            pl.BlockSpec((1, h), lambda i: (0, 0)),
            pl.BlockSpec((h, h), lambda i: (0, 0)),
        ],
        out_specs=[
            pl.BlockSpec((NC, _BK, hh), lambda i: (0, i, 0)),
            pl.BlockSpec((_BK, 1), lambda i: (i, 0)),
        ],
        out_shape=[
            jax.ShapeDtypeStruct((NC, n, hh), _F32),
            jax.ShapeDtypeStruct((n, 1), _F32),
        ],
    )(x, hist, wt, bt, w0)


def _tc_layer(s, g, dinv, bias, w_next):
    _, n, hh = g.shape
    h = NC * hh
    grid = (n // _BK,)

    def body(s_ref, g_ref, dinv_ref, b_ref, w_ref, gn_ref):
        t = jnp.concatenate([s_ref[0] + g_ref[0], s_ref[1] + g_ref[1]], axis=1)
        hcur = jnp.maximum(dinv_ref[...] * t + b_ref[...], 0.0)
        gn_ref[...] = _split(_dot(hcur, w_ref[...]) * dinv_ref[...], hh)

    return pl.pallas_call(
        body,
        grid=grid,
        in_specs=[
            pl.BlockSpec((NC, _BK, hh), lambda i: (0, i, 0)),
            pl.BlockSpec((NC, _BK, hh), lambda i: (0, i, 0)),
            pl.BlockSpec((_BK, 1), lambda i: (i, 0)),
            pl.BlockSpec((1, h), lambda i: (0, 0)),
            pl.BlockSpec((h, h), lambda i: (0, 0)),
        ],
        out_specs=pl.BlockSpec((NC, _BK, hh), lambda i: (0, i, 0)),
        out_shape=jax.ShapeDtypeStruct((NC, n, hh), _F32),
    )(s, g, dinv, bias, w_next)


def _tc_heads(s, g, dinv, bias, wh1a, bh1, wh1b, wp1, bp1, wp2, bp2):
    _, n, hh = g.shape
    h = NC * hh
    hp = wp1.shape[1]
    grid = (n // _BK,)

    def body(s_ref, g_ref, dinv_ref, b_ref, wh1a_ref, bh1_ref, wh1b_ref,
             wp1_ref, bp1_ref, wp2_ref, bp2_ref,
             h_ref, a_ref, bb_ref, p_ref):
        t = jnp.concatenate([s_ref[0] + g_ref[0], s_ref[1] + g_ref[1]], axis=1)
        hcur = jnp.maximum(dinv_ref[...] * t + b_ref[...], 0.0)
        h_ref[...] = hcur
        a_ref[...] = _split(_dot(hcur, wh1a_ref[...]) + bh1_ref[...], hh)
        bb_ref[...] = _split(_dot(hcur, wh1b_ref[...]), hh)
        t2 = jnp.maximum(_dot(hcur, wp1_ref[...]) + bp1_ref[...], 0.0)
        p_ref[...] = jax.nn.sigmoid(_dot(t2, wp2_ref[...]) + bp2_ref[...])

    return pl.pallas_call(
        body,
        grid=grid,
        in_specs=[
            pl.BlockSpec((NC, _BK, hh), lambda i: (0, i, 0)),
            pl.BlockSpec((NC, _BK, hh), lambda i: (0, i, 0)),
            pl.BlockSpec((_BK, 1), lambda i: (i, 0)),
            pl.BlockSpec((1, h), lambda i: (0, 0)),
            pl.BlockSpec((h, h), lambda i: (0, 0)),
            pl.BlockSpec((1, h), lambda i: (0, 0)),
            pl.BlockSpec((h, h), lambda i: (0, 0)),
            pl.BlockSpec((h, hp), lambda i: (0, 0)),
            pl.BlockSpec((1, hp), lambda i: (0, 0)),
            pl.BlockSpec((hp, 1), lambda i: (0, 0)),
            pl.BlockSpec((1, 1), lambda i: (0, 0)),
        ],
        out_specs=[
            pl.BlockSpec((_BK, h), lambda i: (i, 0)),
            pl.BlockSpec((NC, _BK, hh), lambda i: (0, i, 0)),
            pl.BlockSpec((NC, _BK, hh), lambda i: (0, i, 0)),
            pl.BlockSpec((_BK, 1), lambda i: (i, 0)),
        ],
        out_shape=[
            jax.ShapeDtypeStruct((n, h), _F32),
            jax.ShapeDtypeStruct((NC, n, hh), _F32),
            jax.ShapeDtypeStruct((NC, n, hh), _F32),
            jax.ShapeDtypeStruct((n, 1), _F32),
        ],
    )(s, g, dinv, bias, wh1a, bh1, wh1b, wp1, bp1, wp2, bp2)


_BE2 = 3200


def _tc_hier(epre2, w2a, w2b, bh2r):
    _, e2, h = epre2.shape
    hh = h // NC
    k = w2a.shape[0]
    grid = (e2 // _BE2,)

    dn = (((1,), (1,)), ((), ()))

    def body(e_ref, wa_ref, wb_ref, b_ref, he_ref, ho_ref):
        e0 = e_ref[0]
        e1 = e_ref[1]
        wa = wa_ref[...]
        wb = wb_ref[...]
        he_ref[...] = (
            lax.dot_general(wa, jnp.maximum(e0[:, :hh], 0.0), dn,
                            preferred_element_type=_F32)
            + lax.dot_general(wb, jnp.maximum(e1[:, :hh], 0.0), dn,
                              preferred_element_type=_F32) + b_ref[...])
        ho_ref[...] = (
            lax.dot_general(wa, jnp.maximum(e0[:, hh:], 0.0), dn,
                            preferred_element_type=_F32)
            + lax.dot_general(wb, jnp.maximum(e1[:, hh:], 0.0), dn,
                              preferred_element_type=_F32) + b_ref[...])

    return pl.pallas_call(
        body,
        grid=grid,
        in_specs=[
            pl.BlockSpec((NC, _BE2, h), lambda i: (0, i, 0)),
            pl.BlockSpec((k, hh), lambda i: (0, 0)),
            pl.BlockSpec((k, hh), lambda i: (0, 0)),
            pl.BlockSpec((k, 1), lambda i: (0, 0)),
        ],
        out_specs=[
            pl.BlockSpec((k, _BE2), lambda i: (0, i)),
            pl.BlockSpec((k, _BE2), lambda i: (0, i)),
        ],
        out_shape=[
            jax.ShapeDtypeStruct((k, e2), _F32),
            jax.ShapeDtypeStruct((k, e2), _F32),
        ],
    )(epre2, w2a, w2b, bh2r)


def kernel(x, edge_index, params):
    n, d = x.shape
    e = edge_index.shape[1]
    h = params['Wt'].shape[1]
    src = edge_index[0].astype(jnp.int32)
    dst = edge_index[1].astype(jnp.int32)

    chw_a = e // (NS * _AGG_C)
    src_r = src.reshape(NS, chw_a, _AGG_C)
    src4a = jnp.stack([src_r, src_r + n])
    dst3a = dst.reshape(NS, chw_a, _AGG_C)
    half = e // 2
    chw_c = e // (NS * _CMB_C)
    cc2 = _CMB_C // 2
    ilv = lambda v: jnp.concatenate(
        [v[:half].reshape(NS, chw_c, cc2), v[half:].reshape(NS, chw_c, cc2)],
        axis=2)
    src_rc = ilv(src)
    dst_rc = ilv(dst)
    src4c = jnp.stack([src_rc, src_rc + n])
    dst4c = jnp.stack([dst_rc, dst_rc + n])

    row = lambda v: v.reshape(1, -1)
    flat2 = lambda t: t.reshape(NC * n, h // NC)

    hist = _degree_partials(dst, n)
    g0, dinv = _tc_transform(x, hist.T, params['Wt'], row(params['bt']),
                             params['gcn_W0'])
    s0 = _edge_aggregate(flat2(g0), src4a, dst3a)
    g1 = _tc_layer(s0, g0, dinv, row(params['gcn_b0']), params['gcn_W1'])
    s1 = _edge_aggregate(flat2(g1), src4a, dst3a)
    g2 = _tc_layer(s1, g1, dinv, row(params['gcn_b1']), params['gcn_W2'])
    s2 = _edge_aggregate(flat2(g2), src4a, dst3a)
    h3, a, b_tab, perm = _tc_heads(
        s2, g2, dinv, row(params['gcn_b2']),
        params['Wh1'][:h], row(params['bh1']), params['Wh1'][h:],
        params['Wp1'], row(params['bp1']), params['Wp2'], row(params['bp2']))
    epre2 = _edge_combine(flat2(a), flat2(b_tab), src4c, dst4c)
    hh = h // NC
    w2t = params['Wh2'].T
    he, ho = _tc_hier(epre2, w2t[:, :hh], w2t[:, hh:],
                      params['bh2'].reshape(-1, 1))
    hier = jnp.concatenate([he, ho], axis=1).T
    return (h3, hier, perm)


# speedup vs baseline: 1.1370x; 1.1370x over previous
import functools

import jax
import jax.numpy as jnp
from jax import lax
from jax.experimental import pallas as pl
from jax.experimental.pallas import tpu as pltpu
from jax.experimental.pallas import tpu_sc as plsc

NC = 2
NS = 16
NW = NC * NS
LANE = 16

_MESH = plsc.VectorSubcoreMesh(core_axis_name="c", subcore_axis_name="s")
_F32 = jnp.float32
_SC_PARAMS = pltpu.CompilerParams(
    needs_layout_passes=False, use_tc_tiling_on_sc=False)


def _degree_partials(dst_flat, n_nodes):
    e = dst_flat.shape[0]
    per_w = e // NW

    @functools.partial(
        pl.kernel,
        mesh=_MESH,
        out_type=jax.ShapeDtypeStruct((NW, n_nodes), _F32),
        compiler_params=_SC_PARAMS,
        scratch_types=[
            pltpu.VMEM((per_w,), jnp.int32),
            pltpu.VMEM((n_nodes,), _F32),
        ],
    )
    def deg_kernel(dst_hbm, out_hbm, idx_v, acc_v):
        cid = lax.axis_index("c")
        sid = lax.axis_index("s")
        wid = sid * NC + cid
        pltpu.sync_copy(dst_hbm.at[pl.ds(wid * per_w, per_w)], idx_v)
        zero = jnp.zeros((LANE,), _F32)

        def zbody(i, c):
            acc_v[pl.ds(i * LANE, LANE)] = zero
            return c

        lax.fori_loop(0, n_nodes // LANE, zbody, 0)
        ones = jnp.ones((LANE,), _F32)

        def body(i, c):
            idx = idx_v[pl.ds(i * LANE, LANE)]
            plsc.addupdate_scatter(acc_v, [idx], ones)
            return c

        lax.fori_loop(0, per_w // LANE, body, 0)
        pltpu.sync_copy(acc_v, out_hbm.at[wid])

    return deg_kernel(dst_flat)


_AGG_C = 80
_AGG_NB = 5


def _edge_aggregate(g2, src4, dst3):
    n2, hh = g2.shape
    n = n2 // NC
    _, _, chw, c = src4.shape
    rows_t = n // NS
    zrows = rows_t // 5

    @functools.partial(
        pl.kernel,
        mesh=_MESH,
        out_type=jax.ShapeDtypeStruct((NC, n, hh), _F32),
        compiler_params=_SC_PARAMS,
        scratch_types=[
            pltpu.VMEM((chw, c), jnp.int32),
            pltpu.VMEM((chw, c), jnp.int32),
            pltpu.VMEM((_AGG_NB, c, hh), _F32),
            pltpu.VMEM((zrows, hh), _F32),
            pltpu.VMEM_SHARED((n, hh), _F32),
            pltpu.SemaphoreType.DMA,
            pltpu.SemaphoreType.DMA,
            pltpu.SemaphoreType.DMA,
            pltpu.SemaphoreType.DMA,
            pltpu.SemaphoreType.DMA,
        ],
    )
    def agg_kernel(g_hbm, src_hbm, dst_hbm, out_hbm,
                   idx_s, idx_d, buf, zbuf, acc_sh, s0, s1, s2, s3, s4):
        sems = (s0, s1, s2, s3, s4)
        cid = lax.axis_index("c")
        sid = lax.axis_index("s")
        pltpu.sync_copy(src_hbm.at[cid, sid], idx_s)
        pltpu.sync_copy(dst_hbm.at[sid], idx_d)

        zero = jnp.zeros((LANE,), _F32)
        ncol = hh // LANE

        def zbody(i, cc):
            zbuf[i // ncol, pl.ds((i % ncol) * LANE, LANE)] = zero
            return cc

        lax.fori_loop(0, zrows * ncol, zbody, 0)
        for k in range(5):
            pltpu.sync_copy(zbuf, acc_sh.at[pl.ds(sid * rows_t + k * zrows, zrows)])
        plsc.subcore_barrier()

        for b in range(_AGG_NB):
            pltpu.async_copy(g_hbm.at[idx_s.at[b]], buf.at[b], sems[b])

        def outer(j0, cc):
            for b in range(_AGG_NB):
                j = j0 * _AGG_NB + b
                pltpu.make_async_copy(g_hbm.at[idx_s.at[j]], buf.at[b], sems[b]).wait()
                pltpu.sync_copy(buf.at[b], acc_sh.at[idx_d.at[j]], add=True)

                @pl.when(j + _AGG_NB < chw)
                def _():
                    pltpu.async_copy(g_hbm.at[idx_s.at[j + _AGG_NB]], buf.at[b], sems[b])
            return cc

        lax.fori_loop(0, chw // _AGG_NB, outer, 0)
        plsc.subcore_barrier()
        for k in range(5):
            off = sid * rows_t + k * zrows
            pltpu.sync_copy(acc_sh.at[pl.ds(off, zrows)],
                            out_hbm.at[cid, pl.ds(off, zrows)])

    return agg_kernel(g2, src4, dst3)


_CMB_C = 40
_CMB_NB = 5


def _edge_combine(a2, b2, src4, dst4):
    n2, hh = a2.shape
    _, _, chw, c = src4.shape
    per_t = chw * c
    e2 = NS * per_t // 2
    c2 = c // 2
    h = NC * hh

    @functools.partial(
        pl.kernel,
        mesh=_MESH,
        out_type=jax.ShapeDtypeStruct((NC, e2, h), _F32),
        compiler_params=_SC_PARAMS,
        scratch_types=[
            pltpu.VMEM((chw, c), jnp.int32),
            pltpu.VMEM((chw, c), jnp.int32),
            pltpu.VMEM((_CMB_NB, c, hh), _F32),
            pltpu.VMEM((_CMB_NB, c, hh), _F32),
            pltpu.VMEM((_CMB_NB, c2, h), _F32),
            pltpu.SemaphoreType.DMA,
            pltpu.SemaphoreType.DMA,
            pltpu.SemaphoreType.DMA,
            pltpu.SemaphoreType.DMA,
            pltpu.SemaphoreType.DMA,
            pltpu.SemaphoreType.DMA,
            pltpu.SemaphoreType.DMA,
            pltpu.SemaphoreType.DMA,
            pltpu.SemaphoreType.DMA,
            pltpu.SemaphoreType.DMA,
        ],
    )
    def cmb_kernel(a_hbm, b_hbm, src_hbm, dst_hbm, out_hbm,
                   idx_s, idx_d, buf_a, buf_b, buf_o,
                   g0, g1, g2, g3, g4, o0, o1, o2, o3, o4):
        gsem = (g0, g1, g2, g3, g4)
        osem = (o0, o1, o2, o3, o4)
        cid = lax.axis_index("c")
        sid = lax.axis_index("s")
        pltpu.sync_copy(src_hbm.at[cid, sid], idx_s)
        pltpu.sync_copy(dst_hbm.at[cid, sid], idx_d)

        for b in range(_CMB_NB):
            pltpu.async_copy(a_hbm.at[idx_s.at[b]], buf_a.at[b], gsem[b])
            pltpu.async_copy(b_hbm.at[idx_d.at[b]], buf_b.at[b], gsem[b])

        ncol = hh // LANE

        def outer(j0, cc):
            for b in range(_CMB_NB):
                j = j0 * _CMB_NB + b
                out_row = (sid * per_t + j * c) // 2
                pltpu.make_async_copy(a_hbm.at[idx_s.at[j]], buf_a.at[b], gsem[b]).wait()
                pltpu.make_async_copy(b_hbm.at[idx_d.at[j]], buf_b.at[b], gsem[b]).wait()

                @pl.when(j >= _CMB_NB)
                def _():
                    pltpu.make_async_copy(
                        buf_o.at[b],
                        out_hbm.at[cid, pl.ds(out_row - _CMB_NB * c2, c2)],
                        osem[b]).wait()

                def addrow(rr, carry, _b=b):
                    r0 = rr * 2
                    for col in range(2 * ncol):
                        sr = r0 + (1 if col >= ncol else 0)
                        sc = (col % ncol) * LANE
                        buf_o[_b, rr, pl.ds(col * LANE, LANE)] = (
                            buf_a[_b, sr, pl.ds(sc, LANE)]
                            + buf_b[_b, sr, pl.ds(sc, LANE)])
                    return carry

                lax.fori_loop(0, c2, addrow, 0)
                pltpu.async_copy(buf_o.at[b], out_hbm.at[cid, pl.ds(out_row, c2)],
                                 osem[b])

                @pl.when(j + _CMB_NB < chw)
                def _():
                    pltpu.async_copy(a_hbm.at[idx_s.at[j + _CMB_NB]], buf_a.at[b], gsem[b])
                    pltpu.async_copy(b_hbm.at[idx_d.at[j + _CMB_NB]], buf_b.at[b], gsem[b])
            return cc

        lax.fori_loop(0, chw // _CMB_NB, outer, 0)
        for b in range(_CMB_NB):
            j = chw - _CMB_NB + b
            out_row = (sid * per_t + j * c) // 2
            pltpu.make_async_copy(buf_o.at[b], out_hbm.at[cid, pl.ds(out_row, c2)],
                                  osem[b]).wait()

    return cmb_kernel(a2, b2, src4, dst4)


_BK = 2000
_BE = 4000


def _dot(x, w):
    return jnp.dot(x, w, preferred_element_type=_F32)


def _split(t, hh):
    return jnp.stack([t[:, :hh], t[:, hh:]], axis=0)


def _tc_transform(x, hist, wt, bt, w0):
    n, d = x.shape
    h = wt.shape[1]
    hh = h // NC
    grid = (n // _BK,)

    def body(x_ref, hist_ref, wt_ref, bt_ref, w0_ref, g0_ref, dinv_ref):
        deg = 1.0 + jnp.sum(hist_ref[...], axis=1)
        dinv = lax.rsqrt(deg)[:, None]
        h0 = jnp.maximum(_dot(x_ref[...], wt_ref[...]) + bt_ref[...], 0.0)
        g0_ref[...] = _split(_dot(h0, w0_ref[...]) * dinv, hh)
        dinv_ref[...] = dinv

    return pl.pallas_call(
        body,
        grid=grid,
        in_specs=[
            pl.BlockSpec((_BK, d), lambda i: (i, 0)),
            pl.BlockSpec((_BK, NW), lambda i: (i, 0)),
            pl.BlockSpec((d, h), lambda i: (0, 0)),
            pl.BlockSpec((1, h), lambda i: (0, 0)),
            pl.BlockSpec((h, h), lambda i: (0, 0)),
        ],
        out_specs=[
            pl.BlockSpec((NC, _BK, hh), lambda i: (0, i, 0)),
            pl.BlockSpec((_BK, 1), lambda i: (i, 0)),
        ],
        out_shape=[
            jax.ShapeDtypeStruct((NC, n, hh), _F32),
            jax.ShapeDtypeStruct((n, 1), _F32),
        ],
    )(x, hist, wt, bt, w0)


def _tc_layer(s, g, dinv, bias, w_next):
    _, n, hh = g.shape
    h = NC * hh
    grid = (n // _BK,)

    def body(s_ref, g_ref, dinv_ref, b_ref, w_ref, gn_ref):
        t = jnp.concatenate([s_ref[0] + g_ref[0], s_ref[1] + g_ref[1]], axis=1)
        hcur = jnp.maximum(dinv_ref[...] * t + b_ref[...], 0.0)
        gn_ref[...] = _split(_dot(hcur, w_ref[...]) * dinv_ref[...], hh)

    return pl.pallas_call(
        body,
        grid=grid,
        in_specs=[
            pl.BlockSpec((NC, _BK, hh), lambda i: (0, i, 0)),
            pl.BlockSpec((NC, _BK, hh), lambda i: (0, i, 0)),
            pl.BlockSpec((_BK, 1), lambda i: (i, 0)),
            pl.BlockSpec((1, h), lambda i: (0, 0)),
            pl.BlockSpec((h, h), lambda i: (0, 0)),
        ],
        out_specs=pl.BlockSpec((NC, _BK, hh), lambda i: (0, i, 0)),
        out_shape=jax.ShapeDtypeStruct((NC, n, hh), _F32),
    )(s, g, dinv, bias, w_next)


def _tc_heads(s, g, dinv, bias, wh1a, bh1, wh1b, wp1, bp1, wp2, bp2):
    _, n, hh = g.shape
    h = NC * hh
    hp = wp1.shape[1]
    grid = (n // _BK,)

    def body(s_ref, g_ref, dinv_ref, b_ref, wh1a_ref, bh1_ref, wh1b_ref,
             wp1_ref, bp1_ref, wp2_ref, bp2_ref,
             h_ref, a_ref, bb_ref, p_ref):
        t = jnp.concatenate([s_ref[0] + g_ref[0], s_ref[1] + g_ref[1]], axis=1)
        hcur = jnp.maximum(dinv_ref[...] * t + b_ref[...], 0.0)
        h_ref[...] = hcur
        a_ref[...] = _split(_dot(hcur, wh1a_ref[...]) + bh1_ref[...], hh)
        bb_ref[...] = _split(_dot(hcur, wh1b_ref[...]), hh)
        t2 = jnp.maximum(_dot(hcur, wp1_ref[...]) + bp1_ref[...], 0.0)
        p_ref[...] = jax.nn.sigmoid(_dot(t2, wp2_ref[...]) + bp2_ref[...])

    return pl.pallas_call(
        body,
        grid=grid,
        in_specs=[
            pl.BlockSpec((NC, _BK, hh), lambda i: (0, i, 0)),
            pl.BlockSpec((NC, _BK, hh), lambda i: (0, i, 0)),
            pl.BlockSpec((_BK, 1), lambda i: (i, 0)),
            pl.BlockSpec((1, h), lambda i: (0, 0)),
            pl.BlockSpec((h, h), lambda i: (0, 0)),
            pl.BlockSpec((1, h), lambda i: (0, 0)),
            pl.BlockSpec((h, h), lambda i: (0, 0)),
            pl.BlockSpec((h, hp), lambda i: (0, 0)),
            pl.BlockSpec((1, hp), lambda i: (0, 0)),
            pl.BlockSpec((hp, 1), lambda i: (0, 0)),
            pl.BlockSpec((1, 1), lambda i: (0, 0)),
        ],
        out_specs=[
            pl.BlockSpec((_BK, h), lambda i: (i, 0)),
            pl.BlockSpec((NC, _BK, hh), lambda i: (0, i, 0)),
            pl.BlockSpec((NC, _BK, hh), lambda i: (0, i, 0)),
            pl.BlockSpec((_BK, 1), lambda i: (i, 0)),
        ],
        out_shape=[
            jax.ShapeDtypeStruct((n, h), _F32),
            jax.ShapeDtypeStruct((NC, n, hh), _F32),
            jax.ShapeDtypeStruct((NC, n, hh), _F32),
            jax.ShapeDtypeStruct((n, 1), _F32),
        ],
    )(s, g, dinv, bias, wh1a, bh1, wh1b, wp1, bp1, wp2, bp2)


_BE2 = 3200


def _tc_hier(epre2, w2a, w2b, bh2r):
    _, e2, h = epre2.shape
    hh = h // NC
    k = w2a.shape[0]
    grid = (e2 // _BE2,)

    dn = (((1,), (1,)), ((), ()))

    def body(e_ref, wa_ref, wb_ref, b_ref, he_ref, ho_ref):
        e0 = e_ref[0]
        e1 = e_ref[1]
        wa = wa_ref[...]
        wb = wb_ref[...]
        he_ref[...] = (
            lax.dot_general(wa, jnp.maximum(e0[:, :hh], 0.0), dn,
                            preferred_element_type=_F32)
            + lax.dot_general(wb, jnp.maximum(e1[:, :hh], 0.0), dn,
                              preferred_element_type=_F32) + b_ref[...])
        ho_ref[...] = (
            lax.dot_general(wa, jnp.maximum(e0[:, hh:], 0.0), dn,
                            preferred_element_type=_F32)
            + lax.dot_general(wb, jnp.maximum(e1[:, hh:], 0.0), dn,
                              preferred_element_type=_F32) + b_ref[...])

    return pl.pallas_call(
        body,
        grid=grid,
        in_specs=[
            pl.BlockSpec((NC, _BE2, h), lambda i: (0, i, 0)),
            pl.BlockSpec((k, hh), lambda i: (0, 0)),
            pl.BlockSpec((k, hh), lambda i: (0, 0)),
            pl.BlockSpec((k, 1), lambda i: (0, 0)),
        ],
        out_specs=[
            pl.BlockSpec((k, _BE2), lambda i: (0, i)),
            pl.BlockSpec((k, _BE2), lambda i: (0, i)),
        ],
        out_shape=[
            jax.ShapeDtypeStruct((k, e2), _F32),
            jax.ShapeDtypeStruct((k, e2), _F32),
        ],
    )(epre2, w2a, w2b, bh2r)


def kernel(x, edge_index, params):
    n, d = x.shape
    e = edge_index.shape[1]
    h = params['Wt'].shape[1]
    src = edge_index[0].astype(jnp.int32)
    dst = edge_index[1].astype(jnp.int32)

    chw_a = e // (NS * _AGG_C)
    src_r = src.reshape(NS, chw_a, _AGG_C)
    src4a = jnp.stack([src_r, src_r + n])
    dst3a = dst.reshape(NS, chw_a, _AGG_C)
    half = e // 2
    ilv = lambda v: jnp.stack([v[:half], v[half:]], axis=1).reshape(-1)
    src_c = ilv(src)
    dst_c = ilv(dst)
    chw_c = e // (NS * _CMB_C)
    src_rc = src_c.reshape(NS, chw_c, _CMB_C)
    dst_rc = dst_c.reshape(NS, chw_c, _CMB_C)
    src4c = jnp.stack([src_rc, src_rc + n])
    dst4c = jnp.stack([dst_rc, dst_rc + n])

    row = lambda v: v.reshape(1, -1)
    flat2 = lambda t: t.reshape(NC * n, h // NC)

    hist = _degree_partials(dst, n)
    g0, dinv = _tc_transform(x, hist.T, params['Wt'], row(params['bt']),
                             params['gcn_W0'])
    s0 = _edge_aggregate(flat2(g0), src4a, dst3a)
    g1 = _tc_layer(s0, g0, dinv, row(params['gcn_b0']), params['gcn_W1'])
    s1 = _edge_aggregate(flat2(g1), src4a, dst3a)
    g2 = _tc_layer(s1, g1, dinv, row(params['gcn_b1']), params['gcn_W2'])
    s2 = _edge_aggregate(flat2(g2), src4a, dst3a)
    h3, a, b_tab, perm = _tc_heads(
        s2, g2, dinv, row(params['gcn_b2']),
        params['Wh1'][:h], row(params['bh1']), params['Wh1'][h:],
        params['Wp1'], row(params['bp1']), params['Wp2'], row(params['bp2']))
    epre2 = _edge_combine(flat2(a), flat2(b_tab), src4c, dst4c)
    hh = h // NC
    w2t = params['Wh2'].T
    he, ho = _tc_hier(epre2, w2t[:, :hh], w2t[:, hh:],
                      params['bh2'].reshape(-1, 1))
    hier = jnp.concatenate([he, ho], axis=1).T
    return (h3, hier, perm)
